# baseline (speedup 1.0000x reference)
import functools
import math

import jax
import jax.numpy as jnp
from jax.experimental import pallas as pl
from jax.experimental.pallas import tpu as pltpu

B = 16384
D = 128
K = 16
BB = 2048


def _mixture_kernel(value_ref, means_ref, log_stds_ref, log_weights_ref, out_ref):
    v = value_ref[...]
    log_stds = log_stds_ref[...]
    means = means_ref[...]
    lw = log_weights_ref[...]

    inv_var = jnp.exp(-2.0 * log_stds)
    a = means * inv_var
    c = (-0.5 * jnp.sum(means * a, axis=1)
         - jnp.sum(log_stds, axis=1)
         - 0.5 * D * math.log(2.0 * math.pi)
         + lw[0])

    q = jax.lax.dot_general(inv_var, v * v, (((1,), (1,)), ((), ())),
                            preferred_element_type=jnp.float32)
    l = jax.lax.dot_general(a, v, (((1,), (1,)), ((), ())),
                            preferred_element_type=jnp.float32)
    comp = (-0.5 * q + l) + c[:, None]

    m = jnp.max(comp, axis=0, keepdims=True)
    s = jnp.sum(jnp.exp(comp - m), axis=0, keepdims=True)
    out_ref[...] = (m + jnp.log(s))[0]


@functools.partial(jax.jit, static_argnames=())
def kernel(value, means, log_stds, log_weights):
    lw2 = log_weights.reshape(1, K)
    out = pl.pallas_call(
        _mixture_kernel,
        grid=(B // BB,),
        in_specs=[
            pl.BlockSpec((BB, D), lambda i: (i, 0)),
            pl.BlockSpec((K, D), lambda i: (0, 0)),
            pl.BlockSpec((K, D), lambda i: (0, 0)),
            pl.BlockSpec((1, K), lambda i: (0, 0)),
        ],
        out_specs=pl.BlockSpec((BB,), lambda i: (i,)),
        out_shape=jax.ShapeDtypeStruct((B,), jnp.float32),
        compiler_params=pltpu.CompilerParams(
            dimension_semantics=("parallel",),
        ),
    )(value, means, log_stds, lw2)
    return out


# speedup vs baseline: 1.4934x; 1.4934x over previous
import functools
import math

import jax
import jax.numpy as jnp
from jax.experimental import pallas as pl
from jax.experimental.pallas import tpu as pltpu

B = 16384
D = 128
K = 16
BB = 8192


def _mixture_kernel(value_ref, means_ref, log_stds_ref, log_weights_ref, out_ref):
    v = value_ref[...]
    log_stds = log_stds_ref[...]
    means = means_ref[...]
    lw = log_weights_ref[...]

    inv_var = jnp.exp(-2.0 * log_stds)
    a = means * inv_var
    c = (-0.5 * jnp.sum(means * a, axis=1)
         - jnp.sum(log_stds, axis=1)
         - 0.5 * D * math.log(2.0 * math.pi)
         + lw[0])

    q = jax.lax.dot_general(inv_var, v * v, (((1,), (1,)), ((), ())),
                            preferred_element_type=jnp.float32)
    l = jax.lax.dot_general(a, v, (((1,), (1,)), ((), ())),
                            preferred_element_type=jnp.float32)
    comp = (-0.5 * q + l) + c[:, None]

    m = jnp.max(comp, axis=0, keepdims=True)
    s = jnp.sum(jnp.exp(comp - m), axis=0, keepdims=True)
    out_ref[...] = (m + jnp.log(s))[0]


@functools.partial(jax.jit, static_argnames=())
def kernel(value, means, log_stds, log_weights):
    lw2 = log_weights.reshape(1, K)
    out = pl.pallas_call(
        _mixture_kernel,
        grid=(B // BB,),
        in_specs=[
            pl.BlockSpec((BB, D), lambda i: (i, 0)),
            pl.BlockSpec((K, D), lambda i: (0, 0)),
            pl.BlockSpec((K, D), lambda i: (0, 0)),
            pl.BlockSpec((1, K), lambda i: (0, 0)),
        ],
        out_specs=pl.BlockSpec((BB,), lambda i: (i,)),
        out_shape=jax.ShapeDtypeStruct((B,), jnp.float32),
        compiler_params=pltpu.CompilerParams(
            dimension_semantics=("parallel",),
        ),
    )(value, means, log_stds, lw2)
    return out
